# baseline (speedup 1.0000x reference)
import functools

import jax
import jax.numpy as jnp
from jax import lax
from jax.experimental import pallas as pl
from jax.experimental.pallas import tpu as pltpu
from jax.experimental.pallas import tpu_sc as plsc

VOCAB = 1000000
EMBED_DIM = 64
BATCH = 16384
HIST = 50
TOTAL = BATCH * HIST

NUM_CORES = 2
NUM_SUBCORES = 16
NUM_WORKERS = NUM_CORES * NUM_SUBCORES

RPB = 128
CBLK = BATCH // RPB
NBLOCKS = TOTAL // RPB
BPW = NBLOCKS // NUM_WORKERS
PER_WORKER = BPW * RPB
NBUF = 2

_mesh = plsc.VectorSubcoreMesh(core_axis_name="c", subcore_axis_name="s")


@functools.partial(
    pl.kernel,
    out_type=jax.ShapeDtypeStruct((HIST * 8, CBLK, 8, 128), jnp.float32),
    mesh=_mesh,
    scratch_types=[
        pltpu.VMEM((PER_WORKER,), jnp.int32),
        pltpu.VMEM((3, RPB, EMBED_DIM), jnp.float32),
        pltpu.VMEM((NBUF, 8, 8, 129), jnp.float32),
        pltpu.SemaphoreType.DMA,
        pltpu.SemaphoreType.DMA,
    ],
    compiler_params=pltpu.CompilerParams(
        use_tc_tiling_on_sc=False, needs_layout_passes=False
    ),
)
def _embed_lookup(idx_hbm, table_hbm, out_hbm, idx_v, rows_v, tbuf_v, gsem, osem):
    wid = lax.axis_index("s") * NUM_CORES + lax.axis_index("c")
    base_block = wid * BPW

    pltpu.sync_copy(idx_hbm.at[pl.ds(wid * PER_WORKER, PER_WORKER)], idx_v)

    lane = lax.iota(jnp.int32, 16)

    def fire_gather(b, s):
        pltpu.async_copy(
            table_hbm.at[idx_v.at[pl.ds(b * RPB, RPB)]], rows_v.at[s], gsem
        )

    def wait_gather(s):
        pltpu.make_async_copy(
            table_hbm.at[pl.ds(0, RPB)], rows_v.at[s], gsem
        ).wait()

    def out_dst(bg):
        i1 = bg // CBLK
        c = bg % CBLK
        return out_hbm.at[pl.ds(i1 * 8, 8), c]

    def tbuf_src(s):
        return tbuf_v.at[s, pl.ds(0, 8), pl.ds(0, 8), pl.ds(0, 128)]

    def fire_out(b, s):
        pltpu.async_copy(tbuf_src(s), out_dst(base_block + b), osem)

    def wait_out(s):
        pltpu.make_async_copy(tbuf_src(s), out_dst(base_block), osem).wait()

    rvec = [(lane + k * 16) // 8 for k in range(EMBED_DIM // 16)]
    drvec = [(lane + k * 16) % 8 for k in range(EMBED_DIM // 16)]

    def transpose_block(sr, st):
        sv = jnp.full((16,), st, jnp.int32)

        @plsc.parallel_loop(0, RPB, step=1, unroll=16)
        def _(j):
            jv = jnp.full((16,), j, jnp.int32)
            for k in range(EMBED_DIM // 16):
                vec = rows_v[sr, j, pl.ds(k * 16, 16)]
                plsc.store_scatter(tbuf_v, [sv, rvec[k], drvec[k], jv], vec)

    def step(b, sr, st):
        @pl.when(b + 2 < BPW)
        def _():
            fire_gather(b + 2, (sr + 2) % 3)

        wait_gather(sr)

        @pl.when(b >= NBUF)
        def _():
            wait_out(st)

        transpose_block(sr, st)
        fire_out(b, st)

    fire_gather(0, 0)
    fire_gather(1, 1)

    def body(g, carry):
        for u in range(6):
            step(6 * g + u, u % 3, u % 2)
        return carry

    lax.fori_loop(0, BPW // 6, body, 0)

    for b in range(BPW - BPW % 6, BPW):
        step(b, b % 3, b % 2)

    wait_out(0)
    wait_out(1)


def kernel(x, table):
    idx = x.T.reshape(-1).astype(jnp.int32)
    buf = _embed_lookup(idx, table)
    return (
        buf.reshape(HIST, 8, CBLK, 8, 128)
        .transpose(2, 4, 0, 1, 3)
        .reshape(BATCH, HIST, EMBED_DIM)
    )


# speedup vs baseline: 1.0186x; 1.0186x over previous
import functools

import jax
import jax.numpy as jnp
from jax import lax
from jax.experimental import pallas as pl
from jax.experimental.pallas import tpu as pltpu
from jax.experimental.pallas import tpu_sc as plsc

VOCAB = 1000000
EMBED_DIM = 64
BATCH = 16384
HIST = 50
TOTAL = BATCH * HIST

NUM_CORES = 2
NUM_SUBCORES = 16
NUM_WORKERS = NUM_CORES * NUM_SUBCORES

RPB = 128
CBLK = BATCH // RPB
NBLOCKS = TOTAL // RPB
BPW = NBLOCKS // NUM_WORKERS
PER_WORKER = BPW * RPB
NBUF = 2

_mesh = plsc.VectorSubcoreMesh(core_axis_name="c", subcore_axis_name="s")


@functools.partial(
    pl.kernel,
    out_type=jax.ShapeDtypeStruct((HIST * 8, CBLK, 8, 128), jnp.float32),
    mesh=_mesh,
    scratch_types=[
        pltpu.VMEM((PER_WORKER,), jnp.int32),
        pltpu.VMEM((3, RPB, EMBED_DIM), jnp.float32),
        pltpu.VMEM((NBUF, 8, 8, 129), jnp.float32),
        pltpu.SemaphoreType.DMA,
        pltpu.SemaphoreType.DMA,
    ],
    compiler_params=pltpu.CompilerParams(
        use_tc_tiling_on_sc=False, needs_layout_passes=False
    ),
)
def _embed_lookup(idx_hbm, table_hbm, out_hbm, idx_v, rows_v, tbuf_v, gsem, osem):
    wid = lax.axis_index("s") * NUM_CORES + lax.axis_index("c")
    base_block = wid * BPW

    pltpu.sync_copy(idx_hbm.at[pl.ds(wid * PER_WORKER, PER_WORKER)], idx_v)

    lane = lax.iota(jnp.int32, 16)

    def fire_gather(b, s):
        pltpu.async_copy(
            table_hbm.at[idx_v.at[pl.ds(b * RPB, RPB)]], rows_v.at[s], gsem
        )

    def wait_gather(s):
        pltpu.make_async_copy(
            table_hbm.at[pl.ds(0, RPB)], rows_v.at[s], gsem
        ).wait()

    def out_dst(bg):
        i1 = bg // CBLK
        c = bg % CBLK
        return out_hbm.at[pl.ds(i1 * 8, 8), c]

    def tbuf_src(s):
        return tbuf_v.at[s, pl.ds(0, 8), pl.ds(0, 8), pl.ds(0, 128)]

    def fire_out(b, s):
        pltpu.async_copy(tbuf_src(s), out_dst(base_block + b), osem)

    def wait_out(s):
        pltpu.make_async_copy(tbuf_src(s), out_dst(base_block), osem).wait()

    rvec = [(lane + k * 16) // 8 for k in range(EMBED_DIM // 16)]
    drvec = [(lane + k * 16) % 8 for k in range(EMBED_DIM // 16)]

    def transpose_block(sr, st):
        sv = jnp.full((16,), st, jnp.int32)

        @plsc.parallel_loop(0, RPB, step=1, unroll=8)
        def _(j):
            jv = jnp.full((16,), j, jnp.int32)
            for k in range(EMBED_DIM // 16):
                vec = rows_v[sr, j, pl.ds(k * 16, 16)]
                plsc.store_scatter(tbuf_v, [sv, rvec[k], drvec[k], jv], vec)

    def step(b, sr, st):
        @pl.when(b + 2 < BPW)
        def _():
            fire_gather(b + 2, (sr + 2) % 3)

        wait_gather(sr)

        @pl.when(b >= NBUF)
        def _():
            wait_out(st)

        transpose_block(sr, st)
        fire_out(b, st)

    fire_gather(0, 0)
    fire_gather(1, 1)

    def body(g, carry):
        for u in range(6):
            step(6 * g + u, u % 3, u % 2)
        return carry

    lax.fori_loop(0, BPW // 6, body, 0)

    for b in range(BPW - BPW % 6, BPW):
        step(b, b % 3, b % 2)

    wait_out(0)
    wait_out(1)


def kernel(x, table):
    idx = x.T.reshape(-1).astype(jnp.int32)
    buf = _embed_lookup(idx, table)
    return (
        buf.reshape(HIST, 8, CBLK, 8, 128)
        .transpose(2, 4, 0, 1, 3)
        .reshape(BATCH, HIST, EMBED_DIM)
    )
